# Initial kernel scaffold; baseline (speedup 1.0000x reference)
import jax
import jax.numpy as jnp
from jax.experimental import pallas as pl


def kernel(rgb, depth, w1, w2):
    raise NotImplementedError("write your pallas kernel here")


# speedup vs baseline: 1.1137x; 1.1137x over previous
import functools

import jax
import jax.numpy as jnp
from jax.experimental import pallas as pl
from jax.experimental.pallas import tpu as pltpu


def _fused_gate_kernel(rgb_ref, depth_ref, w1rt_ref, w1dt_ref, w2t_ref,
                       out_ref):
    depth = depth_ref[...]

    max_rgb = jnp.max(rgb_ref[...], axis=-1).astype(jnp.float32)
    max_dep = jnp.max(depth, axis=-1).astype(jnp.float32)

    h = (jnp.dot(max_rgb, w1rt_ref[...].astype(jnp.float32),
                 preferred_element_type=jnp.float32)
         + jnp.dot(max_dep, w1dt_ref[...].astype(jnp.float32),
                   preferred_element_type=jnp.float32))
    h = jnp.maximum(h, 0.0)
    attn = jnp.dot(h, w2t_ref[...].astype(jnp.float32),
                   preferred_element_type=jnp.float32)

    gate = jax.nn.sigmoid(attn).astype(out_ref.dtype)
    out_ref[...] = depth * gate[:, :, None]


@functools.partial(jax.jit, static_argnames=("batch_tile",))
def _run(rgb, depth, w1, w2, batch_tile):
    B, Ch, H, W = depth.shape
    Cr = w1.shape[0]
    HW = H * W
    itemsize = jnp.dtype(depth.dtype).itemsize

    rgb_f = rgb.reshape(B, Ch, HW)
    depth_f = depth.reshape(B, Ch, HW)

    w1rt = w1[:, :Ch].T
    w1dt = w1[:, Ch:].T
    w2t = w2.T

    Bt = batch_tile
    while B % Bt:
        Bt -= 1
    grid = (B // Bt,)
    block = (Bt, Ch, HW)
    bmap = lambda b: (b, 0, 0)
    wmap = lambda b: (0, 0)

    out_flat = pl.pallas_call(
        _fused_gate_kernel,
        out_shape=jax.ShapeDtypeStruct((B, Ch, HW), depth.dtype),
        grid=grid,
        in_specs=[
            pl.BlockSpec(block, bmap),
            pl.BlockSpec(block, bmap),
            pl.BlockSpec(w1rt.shape, wmap),
            pl.BlockSpec(w1dt.shape, wmap),
            pl.BlockSpec(w2t.shape, wmap),
        ],
        out_specs=pl.BlockSpec(block, bmap),
        compiler_params=pltpu.CompilerParams(
            dimension_semantics=("parallel",),
            vmem_limit_bytes=64 << 20),
        cost_estimate=pl.CostEstimate(
            flops=3 * B * Ch * HW + 4 * B * Cr * Ch + 2 * B * Ch * Cr,
            transcendentals=B * Ch,
            bytes_accessed=3 * B * Ch * HW * itemsize
            + 2 * (w1.size + w2.size) * itemsize,
        ),
    )(rgb_f, depth_f, w1rt, w1dt, w2t)
    return out_flat.reshape(B, Ch, H, W)


def kernel(rgb, depth, w1, w2):
    return _run(rgb, depth, w1, w2, batch_tile=2)
